# baseline (speedup 1.0000x reference)
import functools

import jax
import jax.numpy as jnp
from jax import lax
from jax.experimental import pallas as pl
from jax.experimental.pallas import tpu as pltpu
from jax.experimental.pallas import tpu_sc as plsc

NOISE = 0.01
L = 16
K = 128
NC = 2
NS = 16
NW = NC * NS


def _sc_segment(N, E, D):
    assert E % K == 0 and D % L == 0
    nchunks = E // K
    NP = ((N + NS * K - 1) // (NS * K)) * (NS * K)
    rows_per_tile = NP // NS
    mesh = plsc.VectorSubcoreMesh(core_axis_name="c", subcore_axis_name="s")

    @functools.partial(
        pl.kernel,
        mesh=mesh,
        out_type=(
            jax.ShapeDtypeStruct((NC, NP, D), jnp.float32),
            jax.ShapeDtypeStruct((NC, NP, L), jnp.float32),
        ),
        scratch_types=[
            pltpu.VMEM((K,), jnp.int32),
            pltpu.VMEM((K,), jnp.int32),
            pltpu.VMEM((K,), jnp.float32),
            pltpu.VMEM((K,), jnp.int32),
            pltpu.VMEM((K, D), jnp.float32),
            pltpu.VMEM((K, L), jnp.float32),
            pltpu.VMEM_SHARED((NP, D), jnp.float32),
            pltpu.VMEM_SHARED((NP, L), jnp.float32),
            pltpu.SemaphoreType.DMA,
        ],
    )
    def k(allocs_hbm, srcs_hbm, dsts_hbm, csi_hbm, out_rows, out_cnt,
          src_v, dst_v, csi_v, idx_v, rows_v, cnt_v, acc_rows, acc_cnt, sem):
        c = lax.axis_index("c")
        s = lax.axis_index("s")
        wid = s * NC + c
        iota = lax.iota(jnp.int32, L)
        onehot0 = jnp.where(iota == 0, 1.0, 0.0).astype(jnp.float32)

        def zero_body(e, carry):
            for blk in range(D // L):
                rows_v[e, pl.ds(blk * L, L)] = jnp.zeros((L,), jnp.float32)
            cnt_v[e, :] = jnp.zeros((L,), jnp.float32)
            return carry
        lax.fori_loop(0, K, zero_body, 0)

        def fill_idx(r0):
            for gg in range(K // L):
                idx_v[pl.ds(gg * L, L)] = r0 + gg * L + iota

        def zinit(j, carry):
            fill_idx(s * rows_per_tile + j * K)
            pltpu.sync_copy(rows_v, acc_rows.at[idx_v])
            pltpu.sync_copy(cnt_v, acc_cnt.at[idx_v])
            return carry
        lax.fori_loop(0, rows_per_tile // K, zinit, 0)
        plsc.subcore_barrier()

        nch = nchunks // NW + jnp.where(wid < nchunks % NW, 1, 0)

        def chunk_body(i, carry):
            base = (wid + i * NW) * K
            pltpu.sync_copy(srcs_hbm.at[pl.ds(base, K)], src_v)
            pltpu.sync_copy(dsts_hbm.at[pl.ds(base, K)], dst_v)
            pltpu.sync_copy(csi_hbm.at[pl.ds(base, K)], csi_v)
            pltpu.async_copy(allocs_hbm.at[src_v], rows_v, sem).wait()

            def group_body(g, gcarry):
                sv = src_v[pl.ds(g * L, L)]
                dv = dst_v[pl.ds(g * L, L)]
                cv = csi_v[pl.ds(g * L, L)]
                m = jnp.where(sv != dv, 1.0, 0.0).astype(jnp.float32)
                w = cv * cv * m
                for lane in range(L):
                    e = g * L + lane
                    wl = w[lane]
                    cnt_v[e, :] = onehot0 * m[lane]
                    for blk in range(D // L):
                        rows_v[e, pl.ds(blk * L, L)] = (
                            rows_v[e, pl.ds(blk * L, L)] * wl)
                return gcarry
            lax.fori_loop(0, K // L, group_body, 0)

            pltpu.sync_copy(rows_v, acc_rows.at[dst_v], add=True)
            pltpu.sync_copy(cnt_v, acc_cnt.at[dst_v], add=True)
            return carry
        lax.fori_loop(0, nch, chunk_body, 0)
        plsc.subcore_barrier()

        def rback(j, carry):
            r0 = s * rows_per_tile + j * K
            fill_idx(r0)
            pltpu.sync_copy(acc_rows.at[idx_v], rows_v)
            pltpu.sync_copy(acc_cnt.at[idx_v], cnt_v)
            pltpu.sync_copy(rows_v, out_rows.at[c, pl.ds(r0, K), :])
            pltpu.sync_copy(cnt_v, out_cnt.at[c, pl.ds(r0, K), :])
            return carry
        lax.fori_loop(0, rows_per_tile // K, rback, 0)

    return k


def _tc_finale(N, D, BN=1000):
    def body(rows_ref, cnt_ref, allocs_ref, scsi_ref, out_ref):
        i = pl.program_id(0)
        interf = rows_ref[0] + rows_ref[1] + NOISE
        cnt = cnt_ref[0, :, 0:1] + cnt_ref[1, :, 0:1]
        s2 = scsi_ref[...] * scsi_ref[...]
        rate = jnp.log2(1.0 + s2 * allocs_ref[...] / interf)
        rate = jnp.where(cnt > 0.0, rate, 0.0)

        @pl.when(i == 0)
        def _():
            out_ref[...] = jnp.zeros((1, 1), jnp.float32)
        out_ref[...] += jnp.sum(rate).reshape(1, 1)

    return pl.pallas_call(
        body,
        grid=(N // BN,),
        in_specs=[
            pl.BlockSpec((NC, BN, D), lambda i: (0, i, 0)),
            pl.BlockSpec((NC, BN, L), lambda i: (0, i, 0)),
            pl.BlockSpec((BN, D), lambda i: (i, 0)),
            pl.BlockSpec((BN, 1), lambda i: (i, 0)),
        ],
        out_specs=pl.BlockSpec((1, 1), lambda i: (0, 0)),
        out_shape=jax.ShapeDtypeStruct((1, 1), jnp.float32),
    )


@jax.jit
def kernel(allocs, node_csi, edge_csi, edge_index):
    N, D = allocs.shape
    E = edge_csi.shape[0]
    rows_p, cnt_p = _sc_segment(N, E, D)(
        allocs, edge_index[0], edge_index[1], edge_csi)
    tot = _tc_finale(N, D)(rows_p, cnt_p, allocs, node_csi)
    return -tot[0, 0] / (N * D)
